# baseline (speedup 1.0000x reference)
import jax
import jax.numpy as jnp
from jax.experimental import pallas as pl
from jax.experimental.pallas import tpu as pltpu

_N = 10000
_D = 128
_BM = 400
_NB = _N // _BM
_NT = 16
_S = _NT * _BM


def _pass1_kernel(x_ref, adj_ref, w1_ref, b1_ref, w2_ref, b2_ref,
                  g2_ref, embud_ref, g1_ref, g2s_ref):
    i = pl.program_id(0)

    @pl.when(i == 0)
    def _g1_phase():
        acc = jnp.dot(x_ref[...], w1_ref[...],
                      preferred_element_type=jnp.float32) + b1_ref[...]
        g1_ref[...] = acc.astype(jnp.bfloat16)

    @pl.when(i >= 1)
    def _stream():
        r = _NB - i
        a = adj_ref[...].astype(jnp.bfloat16)
        h = jnp.dot(a, g1_ref[...], preferred_element_type=jnp.float32)
        h = jnp.maximum(h, 0.0)
        g2 = (jnp.dot(h, w2_ref[...], preferred_element_type=jnp.float32)
              + b2_ref[...]).astype(jnp.bfloat16)
        g2s_ref[pl.ds(r * _BM, _BM), :] = g2
        g2_ref[...] = g2
        rows = jax.lax.broadcasted_iota(jnp.int32, (_N, _D), 0)
        g2m = jnp.where(rows >= r * _BM, g2s_ref[...], jnp.bfloat16(0))
        embud_ref[...] = jnp.dot(a, g2m, preferred_element_type=jnp.float32)


def _lower_sq_kernel(adj_ref, g2_ref, embud_ref, o_ref):
    r = pl.program_id(0) + 1
    a = adj_ref[...].astype(jnp.bfloat16)
    rows = jax.lax.broadcasted_iota(jnp.int32, (_S, _D), 0)
    g2m = jnp.where(rows < r * _BM, g2_ref[0:_S, :], jnp.bfloat16(0))
    o_ref[...] = embud_ref[...] + jnp.dot(
        a, g2m, preferred_element_type=jnp.float32)


def _lower_bot_kernel(adj_ref, g2_ref, embud_ref, embtop_ref,
                      w3_ref, b3_ref, w4_ref, b4_ref, emb_ref, z_ref):
    j = pl.program_id(0)

    @pl.when(j == 0)
    def _row0_pass_through():
        emb_ref[...] = embud_ref[...]

    @pl.when((j >= 1) & (j < _NT))
    def _top_pass_through():
        emb_ref[...] = embtop_ref[...]

    @pl.when(j >= _NT)
    def _bottom():
        a = adj_ref[...].astype(jnp.bfloat16)
        rows = jax.lax.broadcasted_iota(jnp.int32, (_N, _D), 0)
        g2m = jnp.where(rows < j * _BM, g2_ref[...], jnp.bfloat16(0))
        emb_ref[...] = embud_ref[...] + jnp.dot(
            a, g2m, preferred_element_type=jnp.float32)

    emb = emb_ref[...]
    t = jnp.dot(emb, w3_ref[...],
                preferred_element_type=jnp.float32) + b3_ref[...]
    t = jnp.maximum(t, 0.0)
    z_ref[...] = jnp.dot(t, w4_ref[...],
                         preferred_element_type=jnp.float32) + b4_ref[...]


def _rev_map(i):
    return (_NB - jnp.maximum(i, 1), 0)


def _const_map(i):
    return (0, 0)


def kernel(x, Adj_, W1, b1, W2, b2, W3, b3, W4, b4):
    full = lambda r, c: pl.BlockSpec((r, c), _const_map)

    g2, emb_ud = pl.pallas_call(
        _pass1_kernel,
        grid=(1 + _NB,),
        in_specs=[
            full(_N, _D),
            pl.BlockSpec((_BM, _N), _rev_map),
            full(_D, _D), full(1, _D),
            full(_D, _D), full(1, _D),
        ],
        out_specs=[
            pl.BlockSpec((_BM, _D), _rev_map),
            pl.BlockSpec((_BM, _D), _rev_map),
        ],
        out_shape=[
            jax.ShapeDtypeStruct((_N, _D), jnp.bfloat16),
            jax.ShapeDtypeStruct((_N, _D), jnp.float32),
        ],
        scratch_shapes=[
            pltpu.VMEM((_N, _D), jnp.bfloat16),
            pltpu.VMEM((_N, _D), jnp.bfloat16),
        ],
    )(x, Adj_, W1, b1.reshape(1, _D), W2, b2.reshape(1, _D))

    emb_top = pl.pallas_call(
        _lower_sq_kernel,
        grid=(_NT - 1,),
        in_specs=[
            pl.BlockSpec((_BM, _S), lambda j: (j + 1, 0)),
            full(_N, _D),
            pl.BlockSpec((_BM, _D), lambda j: (j + 1, 0)),
        ],
        out_specs=pl.BlockSpec((_BM, _D), lambda j: (j + 1, 0)),
        out_shape=jax.ShapeDtypeStruct((_S, _D), jnp.float32),
    )(Adj_, g2, emb_ud)

    emb, z = pl.pallas_call(
        _lower_bot_kernel,
        grid=(_NB,),
        in_specs=[
            pl.BlockSpec((_BM, _N),
                         lambda j: (jnp.maximum(j, _NT), 0)),
            full(_N, _D),
            pl.BlockSpec((_BM, _D), lambda j: (j, 0)),
            pl.BlockSpec((_BM, _D),
                         lambda j: (jnp.clip(j, 1, _NT - 1), 0)),
            full(_D, _D), full(1, _D),
            full(_D, _D), full(1, _D),
        ],
        out_specs=[
            pl.BlockSpec((_BM, _D), lambda j: (j, 0)),
            pl.BlockSpec((_BM, _D), lambda j: (j, 0)),
        ],
        out_shape=[
            jax.ShapeDtypeStruct((_N, _D), jnp.float32),
            jax.ShapeDtypeStruct((_N, _D), jnp.float32),
        ],
    )(Adj_, g2, emb_ud, emb_top, W3, b3.reshape(1, _D),
      W4, b4.reshape(1, _D))

    return (z, emb)


# speedup vs baseline: 1.0005x; 1.0005x over previous
import jax
import jax.numpy as jnp
from jax.experimental import pallas as pl
from jax.experimental.pallas import tpu as pltpu

_N = 10000
_D = 128
_BM = 400
_NB = _N // _BM
_NT = 16
_S = _NT * _BM


def _pass1_kernel(x_ref, adj_ref, w1_ref, b1_ref, w2_ref, b2_ref,
                  g2_ref, embud_ref, g1_ref, g2s_ref):
    i = pl.program_id(0)

    @pl.when(i == 0)
    def _g1_phase():
        acc = jnp.dot(x_ref[...], w1_ref[...],
                      preferred_element_type=jnp.float32) + b1_ref[...]
        g1_ref[...] = acc.astype(jnp.bfloat16)
        g2s_ref[...] = jnp.zeros((_N, _D), jnp.bfloat16)

    @pl.when(i >= 1)
    def _stream():
        r = _NB - i
        a = adj_ref[...].astype(jnp.bfloat16)
        h = jnp.dot(a, g1_ref[...], preferred_element_type=jnp.float32)
        h = jnp.maximum(h, 0.0)
        g2 = (jnp.dot(h, w2_ref[...], preferred_element_type=jnp.float32)
              + b2_ref[...]).astype(jnp.bfloat16)
        g2s_ref[pl.ds(r * _BM, _BM), :] = g2
        g2_ref[...] = g2
        embud_ref[...] = jnp.dot(a, g2s_ref[...],
                                 preferred_element_type=jnp.float32)


def _lower_sq_kernel(adj_ref, g2_ref, embud_ref, o_ref, g2p_ref):
    j = pl.program_id(0)

    @pl.when(j == 0)
    def _init():
        g2p_ref[...] = jnp.zeros((_S, _D), jnp.bfloat16)

    g2p_ref[pl.ds(j * _BM, _BM), :] = g2_ref[pl.ds(j * _BM, _BM), :]
    a = adj_ref[...].astype(jnp.bfloat16)
    o_ref[...] = embud_ref[...] + jnp.dot(
        a, g2p_ref[...], preferred_element_type=jnp.float32)


def _lower_bot_kernel(adj_ref, g2_ref, embud_ref, embtop_ref,
                      w3_ref, b3_ref, w4_ref, b4_ref, emb_ref, z_ref,
                      g2p_ref):
    j = pl.program_id(0)

    @pl.when(j == 0)
    def _init():
        g2p_ref[...] = jnp.zeros((_N, _D), jnp.bfloat16)

    @pl.when(j >= 1)
    def _extend_prefix():
        g2p_ref[pl.ds((j - 1) * _BM, _BM), :] = (
            g2_ref[pl.ds((j - 1) * _BM, _BM), :])

    @pl.when(j == 0)
    def _row0_pass_through():
        emb_ref[...] = embud_ref[...]

    @pl.when((j >= 1) & (j < _NT))
    def _top_pass_through():
        emb_ref[...] = embtop_ref[...]

    @pl.when(j >= _NT)
    def _bottom():
        a = adj_ref[...].astype(jnp.bfloat16)
        emb_ref[...] = embud_ref[...] + jnp.dot(
            a, g2p_ref[...], preferred_element_type=jnp.float32)

    emb = emb_ref[...]
    t = jnp.dot(emb, w3_ref[...],
                preferred_element_type=jnp.float32) + b3_ref[...]
    t = jnp.maximum(t, 0.0)
    z_ref[...] = jnp.dot(t, w4_ref[...],
                         preferred_element_type=jnp.float32) + b4_ref[...]


def _rev_map(i):
    return (_NB - jnp.maximum(i, 1), 0)


def _const_map(i):
    return (0, 0)


def kernel(x, Adj_, W1, b1, W2, b2, W3, b3, W4, b4):
    full = lambda r, c: pl.BlockSpec((r, c), _const_map)

    g2, emb_ud = pl.pallas_call(
        _pass1_kernel,
        grid=(1 + _NB,),
        in_specs=[
            full(_N, _D),
            pl.BlockSpec((_BM, _N), _rev_map),
            full(_D, _D), full(1, _D),
            full(_D, _D), full(1, _D),
        ],
        out_specs=[
            pl.BlockSpec((_BM, _D), _rev_map),
            pl.BlockSpec((_BM, _D), _rev_map),
        ],
        out_shape=[
            jax.ShapeDtypeStruct((_N, _D), jnp.bfloat16),
            jax.ShapeDtypeStruct((_N, _D), jnp.float32),
        ],
        scratch_shapes=[
            pltpu.VMEM((_N, _D), jnp.bfloat16),
            pltpu.VMEM((_N, _D), jnp.bfloat16),
        ],
    )(x, Adj_, W1, b1.reshape(1, _D), W2, b2.reshape(1, _D))

    emb_top = pl.pallas_call(
        _lower_sq_kernel,
        grid=(_NT - 1,),
        in_specs=[
            pl.BlockSpec((_BM, _S), lambda j: (j + 1, 0)),
            full(_N, _D),
            pl.BlockSpec((_BM, _D), lambda j: (j + 1, 0)),
        ],
        out_specs=pl.BlockSpec((_BM, _D), lambda j: (j + 1, 0)),
        out_shape=jax.ShapeDtypeStruct((_S, _D), jnp.float32),
        scratch_shapes=[pltpu.VMEM((_S, _D), jnp.bfloat16)],
    )(Adj_, g2, emb_ud)

    emb, z = pl.pallas_call(
        _lower_bot_kernel,
        grid=(_NB,),
        in_specs=[
            pl.BlockSpec((_BM, _N),
                         lambda j: (jnp.maximum(j, _NT), 0)),
            full(_N, _D),
            pl.BlockSpec((_BM, _D), lambda j: (j, 0)),
            pl.BlockSpec((_BM, _D),
                         lambda j: (jnp.clip(j, 1, _NT - 1), 0)),
            full(_D, _D), full(1, _D),
            full(_D, _D), full(1, _D),
        ],
        out_specs=[
            pl.BlockSpec((_BM, _D), lambda j: (j, 0)),
            pl.BlockSpec((_BM, _D), lambda j: (j, 0)),
        ],
        out_shape=[
            jax.ShapeDtypeStruct((_N, _D), jnp.float32),
            jax.ShapeDtypeStruct((_N, _D), jnp.float32),
        ],
        scratch_shapes=[pltpu.VMEM((_N, _D), jnp.bfloat16)],
    )(Adj_, g2, emb_ud, emb_top, W3, b3.reshape(1, _D),
      W4, b4.reshape(1, _D))

    return (z, emb)


# speedup vs baseline: 1.4020x; 1.4014x over previous
import jax
import jax.numpy as jnp
from jax.experimental import pallas as pl
from jax.experimental.pallas import tpu as pltpu

_N = 10000
_D = 128
_BM = 400
_NB = _N // _BM


def _fused_kernel(x_ref, adj_ref, w1_ref, b1_ref, w2_ref, b2_ref,
                  w3_ref, b3_ref, w4_ref, b4_ref,
                  emb_ref, z_ref, g1_ref, g2_ref):
    i = pl.program_id(0)

    @pl.when(i == 0)
    def _g1_phase():
        acc = jnp.dot(x_ref[...], w1_ref[...],
                      preferred_element_type=jnp.float32) + b1_ref[...]
        g1_ref[...] = acc.astype(jnp.bfloat16)

    @pl.when((i >= 1) & (i <= _NB))
    def _pass1_phase():
        a = adj_ref[...].astype(jnp.bfloat16)
        h = jnp.dot(a, g1_ref[...], preferred_element_type=jnp.float32)
        h = jnp.maximum(h, 0.0)
        g2 = jnp.dot(h, w2_ref[...],
                     preferred_element_type=jnp.float32) + b2_ref[...]
        g2_ref[pl.ds((i - 1) * _BM, _BM), :] = g2.astype(jnp.bfloat16)

    @pl.when(i > _NB)
    def _pass2_phase():
        a = adj_ref[...].astype(jnp.bfloat16)
        emb = jnp.dot(a, g2_ref[...], preferred_element_type=jnp.float32)
        emb_ref[...] = emb
        t = jnp.dot(emb, w3_ref[...],
                    preferred_element_type=jnp.float32) + b3_ref[...]
        t = jnp.maximum(t, 0.0)
        z_ref[...] = jnp.dot(t, w4_ref[...],
                             preferred_element_type=jnp.float32) + b4_ref[...]


def _adj_map(i):
    return (jnp.where(i <= _NB, jnp.maximum(i - 1, 0), i - 1 - _NB), 0)


def _out_map(i):
    return (jnp.clip(i - 1 - _NB, 0, _NB - 1), 0)


def _const_map(i):
    return (0, 0)


def kernel(x, Adj_, W1, b1, W2, b2, W3, b3, W4, b4):
    full = lambda r, c: pl.BlockSpec((r, c), _const_map)
    emb, z = pl.pallas_call(
        _fused_kernel,
        grid=(1 + 2 * _NB,),
        in_specs=[
            full(_N, _D),
            pl.BlockSpec((_BM, _N), _adj_map),
            full(_D, _D), full(1, _D),
            full(_D, _D), full(1, _D),
            full(_D, _D), full(1, _D),
            full(_D, _D), full(1, _D),
        ],
        out_specs=[
            pl.BlockSpec((_BM, _D), _out_map),
            pl.BlockSpec((_BM, _D), _out_map),
        ],
        out_shape=[
            jax.ShapeDtypeStruct((_N, _D), jnp.float32),
            jax.ShapeDtypeStruct((_N, _D), jnp.float32),
        ],
        scratch_shapes=[
            pltpu.VMEM((_N, _D), jnp.bfloat16),
            pltpu.VMEM((_N, _D), jnp.bfloat16),
        ],
    )(x, Adj_, W1, b1.reshape(1, _D), W2, b2.reshape(1, _D),
      W3, b3.reshape(1, _D), W4, b4.reshape(1, _D))
    return (z, emb)
